# baseline (speedup 1.0000x reference)
import functools

import jax
import jax.numpy as jnp
from jax import lax
from jax.experimental import pallas as pl
from jax.experimental.pallas import tpu as pltpu
from jax.experimental.pallas import tpu_sc as plsc

_L = 16
_R = 128
_CR = 8
_NW = 32
_OW = 128
_NB = 3


@functools.lru_cache(maxsize=None)
def _build(B: int, E: int):
    CM = _CR * _R
    MR = _OW // _L
    CMR = CM // MR
    NCH = -(-B // CM)
    PW = -(-NCH // _NW)
    assert PW >= 4
    mesh = plsc.VectorSubcoreMesh(core_axis_name="c", subcore_axis_name="s")

    @functools.partial(
        pl.kernel,
        mesh=mesh,
        out_type=jax.ShapeDtypeStruct((B // MR, _OW), jnp.float32),
        scratch_types=[
            pltpu.VMEM((_NB, CM), jnp.int32),
            pltpu.VMEM((_NB, CM, _L), jnp.float32),
            pltpu.VMEM((_NB, CMR, _OW), jnp.float32),
            pltpu.SemaphoreType.DMA((_NB,)),
            pltpu.SemaphoreType.DMA((_NB,)),
            pltpu.SemaphoreType.DMA((_NB,)),
        ],
        compiler_params=pltpu.CompilerParams(use_tc_tiling_on_sc=False),
    )
    def k(idx_hbm, table_hbm, out_hbm, idxb, rows, obuf, sem_i, sem_g, sem_o):
        w = lax.axis_index("s") * 2 + lax.axis_index("c")
        lane = lax.iota(jnp.int32, _L)
        perm = ((lane & 3) << 2) | (lane >> 2)

        def chunk_of(j):
            return jnp.minimum(w + j * _NW, NCH - 1)

        def fire_gathers(buf, j):
            for t in range(_CR):
                pltpu.async_copy(
                    table_hbm.at[idxb.at[buf, pl.ds(t * _R, _R)]],
                    rows.at[buf, pl.ds(t * _R, _R)],
                    sem_g.at[buf],
                )

        def fire_idx_load(buf, j):
            base_m = jnp.minimum(chunk_of(j) * CM, B - CM)
            pltpu.async_copy(
                idx_hbm.at[pl.ds(base_m, CM)], idxb.at[buf], sem_i.at[buf]
            )

        for p in (0, 1):
            base_m = jnp.minimum(chunk_of(p) * CM, B - CM)
            pltpu.sync_copy(idx_hbm.at[pl.ds(base_m, CM)], idxb.at[p])
            fire_gathers(p, p)
        fire_idx_load(2, 2)

        def chunk_body(j, carry):
            b = lax.rem(j, _NB)
            fb = lax.rem(j + 2, _NB)

            @pl.when(j + 2 < PW)
            def _():
                pltpu.make_async_copy(
                    idx_hbm.at[pl.ds(0, CM)], idxb.at[fb], sem_i.at[fb]
                ).wait()
                fire_gathers(fb, j + 2)

            pltpu.make_async_copy(
                table_hbm.at[pl.ds(0, CM)], rows.at[b], sem_g.at[b]
            ).wait()

            @pl.when(j >= _NB)
            def _():
                pltpu.make_async_copy(
                    obuf.at[b], out_hbm.at[pl.ds(0, CMR)], sem_o.at[b]
                ).wait()

            for c8 in range(MR):
                def mat_body(r, carry2, c8=c8):
                    v = rows[b, r * MR + c8, :]
                    vt = v[perm]
                    obuf[b, r, pl.ds(c8 * _L, _L)] = (v - vt) * 0.5
                    return carry2

                lax.fori_loop(0, CMR, mat_body, 0, unroll=4)

            base_r = jnp.minimum(chunk_of(j) * CMR, B // MR - CMR)
            pltpu.async_copy(
                obuf.at[b], out_hbm.at[pl.ds(base_r, CMR)], sem_o.at[b]
            )

            @pl.when(j + _NB < PW)
            def _():
                fire_idx_load(b, j + _NB)

            return carry

        lax.fori_loop(0, PW, chunk_body, 0)

        for j in range(PW - _NB, PW):
            pltpu.make_async_copy(
                obuf.at[j % _NB], out_hbm.at[pl.ds(0, CMR)], sem_o.at[j % _NB]
            ).wait()

    return k


def kernel(edge_indices, omega_params):
    B = edge_indices.shape[0]
    E = omega_params.shape[0]
    table = omega_params.reshape(E, _L)
    out = _build(B, E)(edge_indices.astype(jnp.int32), table)
    return out.reshape(B, 4, 4)


# speedup vs baseline: 2.7595x; 2.7595x over previous
import functools

import jax
import jax.numpy as jnp
from jax import lax
from jax.experimental import pallas as pl
from jax.experimental.pallas import tpu as pltpu
from jax.experimental.pallas import tpu_sc as plsc

_L = 16
_R = 128
_CR = 8
_NW = 32
_OW = 128
_NB = 3


@functools.lru_cache(maxsize=None)
def _build(B: int, E: int):
    CM = _CR * _R
    MR = _OW // _L
    CMR = CM // MR
    NCH = -(-B // CM)
    PW = -(-NCH // _NW)
    assert PW >= 4
    mesh = plsc.VectorSubcoreMesh(core_axis_name="c", subcore_axis_name="s")

    @functools.partial(
        pl.kernel,
        mesh=mesh,
        out_type=jax.ShapeDtypeStruct((B, _L), jnp.float32),
        scratch_types=[
            pltpu.VMEM((_NB, CM), jnp.int32),
            pltpu.VMEM((_NB, CM, _L), jnp.float32),
            pltpu.VMEM((_NB, CM, _L), jnp.float32),
            pltpu.SemaphoreType.DMA((_NB,)),
            pltpu.SemaphoreType.DMA((_NB,)),
            pltpu.SemaphoreType.DMA((_NB,)),
        ],
        compiler_params=pltpu.CompilerParams(use_tc_tiling_on_sc=False),
    )
    def k(idx_hbm, table_hbm, out_hbm, idxb, rows, obuf, sem_i, sem_g, sem_o):
        w = lax.axis_index("s") * 2 + lax.axis_index("c")
        lane = lax.iota(jnp.int32, _L)
        perm = ((lane & 3) << 2) | (lane >> 2)

        def chunk_of(j):
            return jnp.minimum(w + j * _NW, NCH - 1)

        def fire_gathers(buf, j):
            for t in range(_CR):
                pltpu.async_copy(
                    table_hbm.at[idxb.at[buf, pl.ds(t * _R, _R)]],
                    rows.at[buf, pl.ds(t * _R, _R)],
                    sem_g.at[buf],
                )

        def fire_idx_load(buf, j):
            base_m = jnp.minimum(chunk_of(j) * CM, B - CM)
            pltpu.async_copy(
                idx_hbm.at[pl.ds(base_m, CM)], idxb.at[buf], sem_i.at[buf]
            )

        for p in (0, 1):
            base_m = jnp.minimum(chunk_of(p) * CM, B - CM)
            pltpu.sync_copy(idx_hbm.at[pl.ds(base_m, CM)], idxb.at[p])
            fire_gathers(p, p)
        fire_idx_load(2, 2)

        def chunk_body(j, carry):
            b = lax.rem(j, _NB)
            fb = lax.rem(j + 2, _NB)

            @pl.when(j + 2 < PW)
            def _():
                pltpu.make_async_copy(
                    idx_hbm.at[pl.ds(0, CM)], idxb.at[fb], sem_i.at[fb]
                ).wait()
                fire_gathers(fb, j + 2)

            pltpu.make_async_copy(
                table_hbm.at[pl.ds(0, CM)], rows.at[b], sem_g.at[b]
            ).wait()

            @pl.when(j >= _NB)
            def _():
                pltpu.make_async_copy(
                    obuf.at[b], out_hbm.at[pl.ds(0, CM)], sem_o.at[b]
                ).wait()

            def mat_body(i, carry2):
                v = rows[b, i, :]
                vt = v[perm]
                obuf[b, i, :] = (v - vt) * 0.5
                return carry2

            lax.fori_loop(0, CM, mat_body, 0, unroll=4)

            base_m = jnp.minimum(chunk_of(j) * CM, B - CM)
            pltpu.async_copy(
                obuf.at[b], out_hbm.at[pl.ds(base_m, CM)], sem_o.at[b]
            )

            @pl.when(j + _NB < PW)
            def _():
                fire_idx_load(b, j + _NB)

            return carry

        lax.fori_loop(0, PW, chunk_body, 0)

        for j in range(PW - _NB, PW):
            pltpu.make_async_copy(
                obuf.at[j % _NB], out_hbm.at[pl.ds(0, CM)], sem_o.at[j % _NB]
            ).wait()

    return k


def kernel(edge_indices, omega_params):
    B = edge_indices.shape[0]
    E = omega_params.shape[0]
    table = omega_params.reshape(E, _L)
    out = _build(B, E)(edge_indices.astype(jnp.int32), table)
    return out.reshape(B, 4, 4)


# speedup vs baseline: 3.2010x; 1.1600x over previous
import functools

import jax
import jax.numpy as jnp
from jax import lax
from jax.experimental import pallas as pl
from jax.experimental.pallas import tpu as pltpu
from jax.experimental.pallas import tpu_sc as plsc

_L = 16
_R = 128
_CR = 8
_NW = 32
_OW = 128
_NB = 3


@functools.lru_cache(maxsize=None)
def _build(B: int, E: int):
    CM = _CR * _R
    MR = _OW // _L
    CMR = CM // MR
    NCH = -(-B // CM)
    PW = -(-NCH // _NW)
    assert PW >= 4
    mesh = plsc.VectorSubcoreMesh(core_axis_name="c", subcore_axis_name="s")

    @functools.partial(
        pl.kernel,
        mesh=mesh,
        out_type=jax.ShapeDtypeStruct((B, _L), jnp.float32),
        scratch_types=[
            pltpu.VMEM((_NB, CM), jnp.int32),
            pltpu.VMEM((_NB, CM, _L), jnp.float32),
            pltpu.VMEM((_NB, CM, _L), jnp.float32),
            pltpu.SemaphoreType.DMA((_NB,)),
            pltpu.SemaphoreType.DMA((_NB,)),
            pltpu.SemaphoreType.DMA((_NB,)),
        ],
        compiler_params=pltpu.CompilerParams(use_tc_tiling_on_sc=False),
    )
    def k(idx_hbm, table_hbm, out_hbm, idxb, rows, obuf, sem_i, sem_g, sem_o):
        w = lax.axis_index("s") * 2 + lax.axis_index("c")
        lane = lax.iota(jnp.int32, _L)
        perm = ((lane & 3) << 2) | (lane >> 2)

        def chunk_of(j):
            return jnp.minimum(w + j * _NW, NCH - 1)

        def fire_gathers(buf, j):
            for t in range(_CR):
                pltpu.async_copy(
                    table_hbm.at[idxb.at[buf, pl.ds(t * _R, _R)]],
                    rows.at[buf, pl.ds(t * _R, _R)],
                    sem_g.at[buf],
                )

        def fire_idx_load(buf, j):
            base_m = jnp.minimum(chunk_of(j) * CM, B - CM)
            pltpu.async_copy(
                idx_hbm.at[pl.ds(base_m, CM)], idxb.at[buf], sem_i.at[buf]
            )

        for p in (0, 1):
            base_m = jnp.minimum(chunk_of(p) * CM, B - CM)
            pltpu.sync_copy(idx_hbm.at[pl.ds(base_m, CM)], idxb.at[p])
            fire_gathers(p, p)
        fire_idx_load(2, 2)

        def chunk_body(j, carry):
            b = lax.rem(j, _NB)
            fb = lax.rem(j + 2, _NB)

            @pl.when(j + 2 < PW)
            def _():
                pltpu.make_async_copy(
                    idx_hbm.at[pl.ds(0, CM)], idxb.at[fb], sem_i.at[fb]
                ).wait()
                fire_gathers(fb, j + 2)

            pltpu.make_async_copy(
                table_hbm.at[pl.ds(0, CM)], rows.at[b], sem_g.at[b]
            ).wait()

            @pl.when(j >= _NB)
            def _():
                pltpu.make_async_copy(
                    obuf.at[b], out_hbm.at[pl.ds(0, CM)], sem_o.at[b]
                ).wait()

            @functools.partial(plsc.parallel_loop, 0, CM, unroll=8)
            def _(i):
                v = rows[b, i, :]
                vt = v[perm]
                obuf[b, i, :] = (v - vt) * 0.5

            base_m = jnp.minimum(chunk_of(j) * CM, B - CM)
            pltpu.async_copy(
                obuf.at[b], out_hbm.at[pl.ds(base_m, CM)], sem_o.at[b]
            )

            @pl.when(j + _NB < PW)
            def _():
                fire_idx_load(b, j + _NB)

            return carry

        lax.fori_loop(0, PW, chunk_body, 0)

        for j in range(PW - _NB, PW):
            pltpu.make_async_copy(
                obuf.at[j % _NB], out_hbm.at[pl.ds(0, CM)], sem_o.at[j % _NB]
            ).wait()

    return k


def kernel(edge_indices, omega_params):
    B = edge_indices.shape[0]
    E = omega_params.shape[0]
    table = omega_params.reshape(E, _L)
    out = _build(B, E)(edge_indices.astype(jnp.int32), table)
    return out.reshape(B, 4, 4)
